# baseline (speedup 1.0000x reference)
import functools

import jax
import jax.numpy as jnp
from jax import lax
from jax.experimental import pallas as pl
from jax.experimental.pallas import tpu as pltpu
from jax.experimental.pallas import tpu_sc as plsc

NUM_ROWS = 1_000_000
DIM = 64
B, T = 4096, 200
B_TOTAL = B * T

NC, NS = 2, 16
NW = NC * NS
IDX_W = 128
K = 8
CHUNK = K * IDX_W
B_PER_W = B_TOTAL // NW
N_STEPS = B_PER_W // CHUNK

_mesh = plsc.VectorSubcoreMesh(core_axis_name="c", subcore_axis_name="s")


@functools.partial(
    pl.kernel,
    mesh=_mesh,
    compiler_params=pltpu.CompilerParams(use_tc_tiling_on_sc=False),
    out_type=jax.ShapeDtypeStruct((B_TOTAL, DIM), jnp.float32),
    scratch_types=[
        pltpu.VMEM((K, IDX_W), jnp.int32),
        pltpu.VMEM((CHUNK, DIM), jnp.float32),
        pltpu.SemaphoreType.DMA,
    ],
)
def _gather_kernel(table_hbm, idx_hbm, out_hbm, idx_v, rows_v, sem):
    wid = lax.axis_index("s") * NC + lax.axis_index("c")
    base = wid * B_PER_W

    def step(i, carry):
        off = pl.multiple_of(base + i * CHUNK, CHUNK)
        row = pl.multiple_of(off // IDX_W, K)
        pltpu.sync_copy(idx_hbm.at[pl.ds(row, K)], idx_v)
        copies = [
            pltpu.async_copy(
                table_hbm.at[idx_v.at[j]],
                rows_v.at[pl.ds(j * IDX_W, IDX_W)],
                sem,
            )
            for j in range(K)
        ]
        for c in copies:
            c.wait()
        pltpu.sync_copy(rows_v, out_hbm.at[pl.ds(off, CHUNK)])
        return carry

    lax.fori_loop(0, N_STEPS, step, 0)


def kernel(inputs, table):
    idx = inputs.reshape(B_TOTAL // IDX_W, IDX_W).astype(jnp.int32)
    out = _gather_kernel(table, idx)
    return out.reshape(B, T, DIM)


# speedup vs baseline: 1.0160x; 1.0160x over previous
import functools

import jax
import jax.numpy as jnp
from jax import lax
from jax.experimental import pallas as pl
from jax.experimental.pallas import tpu as pltpu
from jax.experimental.pallas import tpu_sc as plsc

NUM_ROWS = 1_000_000
DIM = 64
B, T = 4096, 200
B_TOTAL = B * T

NC, NS = 2, 16
NW = NC * NS
IDX_W = 128
K = 5
CHUNK = K * IDX_W
B_PER_W = B_TOTAL // NW
IDX_ROWS = B_PER_W // IDX_W
N_STEPS = B_PER_W // CHUNK
N_HALF = N_STEPS // 2

_mesh = plsc.VectorSubcoreMesh(core_axis_name="c", subcore_axis_name="s")


@functools.partial(
    pl.kernel,
    mesh=_mesh,
    compiler_params=pltpu.CompilerParams(use_tc_tiling_on_sc=False),
    out_type=jax.ShapeDtypeStruct((B_TOTAL, DIM), jnp.float32),
    scratch_types=[
        pltpu.VMEM((IDX_ROWS, IDX_W), jnp.int32),
        pltpu.VMEM((CHUNK, DIM), jnp.float32),
        pltpu.VMEM((CHUNK, DIM), jnp.float32),
        pltpu.SemaphoreType.DMA,
        pltpu.SemaphoreType.DMA,
        pltpu.SemaphoreType.DMA,
        pltpu.SemaphoreType.DMA,
    ],
)
def _gather_kernel(table_hbm, idx_hbm, out_hbm, idx_all, rows_a, rows_b,
                   sem_ga, sem_gb, sem_oa, sem_ob):
    wid = lax.axis_index("s") * NC + lax.axis_index("c")
    base = pl.multiple_of(wid * B_PER_W, CHUNK)
    idx_row0 = pl.multiple_of(wid * IDX_ROWS, 8)

    pltpu.sync_copy(idx_hbm.at[pl.ds(idx_row0, IDX_ROWS)], idx_all)

    def fire_gathers(s, rows, sem):
        for j in range(K):
            pltpu.async_copy(
                table_hbm.at[idx_all.at[s * K + j]],
                rows.at[pl.ds(j * IDX_W, IDX_W)],
                sem,
            )

    def wait_gathers(rows, sem):
        pltpu.make_async_copy(table_hbm.at[pl.ds(0, CHUNK)], rows, sem).wait()

    def fire_out(s, rows, sem):
        pltpu.async_copy(rows, out_hbm.at[pl.ds(base + s * CHUNK, CHUNK)], sem)

    def wait_out(rows, sem):
        pltpu.make_async_copy(rows, out_hbm.at[pl.ds(0, CHUNK)], sem).wait()

    fire_gathers(0, rows_a, sem_ga)

    def body(h, carry):
        s = h * 2
        wait_gathers(rows_a, sem_ga)

        @pl.when(h >= 1)
        def _():
            wait_out(rows_b, sem_ob)

        fire_gathers(s + 1, rows_b, sem_gb)
        fire_out(s, rows_a, sem_oa)

        wait_gathers(rows_b, sem_gb)

        @pl.when(h <= N_HALF - 2)
        def _():
            wait_out(rows_a, sem_oa)
            fire_gathers(s + 2, rows_a, sem_ga)

        fire_out(s + 1, rows_b, sem_ob)
        return carry

    lax.fori_loop(0, N_HALF, body, 0)
    wait_out(rows_a, sem_oa)
    wait_out(rows_b, sem_ob)


def kernel(inputs, table):
    idx = inputs.reshape(B_TOTAL // IDX_W, IDX_W).astype(jnp.int32)
    out = _gather_kernel(table, idx)
    return out.reshape(B, T, DIM)
